# baseline (speedup 1.0000x reference)
import functools

import jax
import jax.numpy as jnp
from jax import lax
from jax.experimental import pallas as pl
from jax.experimental.pallas import tpu as pltpu
from jax.experimental.pallas import tpu_sc as plsc

N_ = 10000
E_ = 160000
C_ = 4
H_ = 128
NCLS = 2
L_ = 2

NSC = 2
NTILE = 16
NW = NSC * NTILE
CH = 128
E_PAD = 163840
EPW = E_PAD // NW
NCHUNK = EPW // CH
CHA = 64
NCHUNKA = EPW // CHA
NPAD = 10240
NROW = NPAD // NTILE

BE = 4096
BN = 2000
F32 = jnp.float32


def _k4_affine(ea, a_ref, c0_ref):
    e = c0_ref[...]
    for c in range(C_):
        e = e + ea[:, c:c + 1] * a_ref[c:c + 1, :]
    return e


def _enc_body(ea_ref, a_ref, c0_ref, e_ref):
    i = pl.program_id(0)
    e = _k4_affine(ea_ref[...], a_ref, c0_ref)
    rows = lax.broadcasted_iota(jnp.int32, (BE, 1), 0) + i * BE
    valid = rows < E_
    e_ref[...] = jnp.where(valid, e, -1e30)


def _h0_body(x_ref, w_ref, b_ref, h_ref):
    h_ref[...] = x_ref[...] * w_ref[...] + b_ref[...]


NBLK = N_ // BN


def _mlpbn_phase0(h_ref, p0_ref, p1_ref, w1_ref, b1_ref, w2_ref, b2_ref,
                  z2_scr, s1_acc, s2_acc, st_scr):
    j = pl.program_id(1)
    z = h_ref[...] + p0_ref[...] + p1_ref[...]
    y = jnp.maximum(jnp.dot(z, w1_ref[...], preferred_element_type=F32, precision=lax.Precision.HIGHEST) + b1_ref[...], 0.0)
    z2 = jnp.dot(y, w2_ref[...], preferred_element_type=F32, precision=lax.Precision.HIGHEST) + b2_ref[...]
    z2_scr[pl.ds(j * BN, BN), :] = z2

    @pl.when(j == 0)
    def _():
        s1_acc[...] = jnp.zeros_like(s1_acc)
        s2_acc[...] = jnp.zeros_like(s2_acc)

    s1_acc[...] = s1_acc[...] + jnp.sum(z2, axis=0, keepdims=True)
    s2_acc[...] = s2_acc[...] + jnp.sum(z2 * z2, axis=0, keepdims=True)

    @pl.when(j == NBLK - 1)
    def _():
        mu = s1_acc[...] / N_
        var = s2_acc[...] / N_ - mu * mu
        st_scr[0:1, :] = mu
        st_scr[1:2, :] = lax.rsqrt(var + 1e-5)


def _mlpbn_norm(h_ref, g_ref, be_ref, z2_scr, st_scr):
    j = pl.program_id(1)
    z2 = z2_scr[pl.ds(j * BN, BN), :]
    zn = (z2 - st_scr[0:1, :]) * st_scr[1:2, :] * g_ref[...] + be_ref[...]
    return h_ref[...] + jnp.maximum(zn, 0.0)


def _mlpbn_body(h_ref, p0_ref, p1_ref, w1_ref, b1_ref, w2_ref, b2_ref,
                g_ref, be_ref, ho_ref, z2_scr, s1_acc, s2_acc, st_scr):
    i = pl.program_id(0)

    @pl.when(i == 0)
    def _():
        _mlpbn_phase0(h_ref, p0_ref, p1_ref, w1_ref, b1_ref, w2_ref, b2_ref,
                      z2_scr, s1_acc, s2_acc, st_scr)

    @pl.when(i == 1)
    def _():
        ho_ref[...] = _mlpbn_norm(h_ref, g_ref, be_ref, z2_scr, st_scr)


def _mlpbn_cls_body(h_ref, p0_ref, p1_ref, w1_ref, b1_ref, w2_ref, b2_ref,
                    g_ref, be_ref, wa_ref, wb_ref, ho_ref, hs_ref, hd_ref,
                    z2_scr, s1_acc, s2_acc, st_scr):
    i = pl.program_id(0)

    @pl.when(i == 0)
    def _():
        _mlpbn_phase0(h_ref, p0_ref, p1_ref, w1_ref, b1_ref, w2_ref, b2_ref,
                      z2_scr, s1_acc, s2_acc, st_scr)

    @pl.when(i == 1)
    def _():
        ho = _mlpbn_norm(h_ref, g_ref, be_ref, z2_scr, st_scr)
        ho_ref[...] = ho
        hs_ref[...] = jnp.dot(ho, wa_ref[...], preferred_element_type=F32, precision=lax.Precision.HIGHEST)
        hd_ref[...] = jnp.dot(ho, wb_ref[...], preferred_element_type=F32, precision=lax.Precision.HIGHEST)


def _head_body(s_ref, ea_ref, a2_ref, c02_ref, w_ref, b_ref, o_ref):
    e1 = _k4_affine(ea_ref[...], a2_ref, c02_ref)
    t = jnp.maximum(s_ref[...] + e1, 0.0)
    o_ref[...] = jnp.dot(t, w_ref[...],
                         preferred_element_type=F32, precision=lax.Precision.HIGHEST) + b_ref[...]


def _full2d(shape):
    return pl.BlockSpec(shape, lambda i: (0, 0))


def _enc_call(eap, a, c0):
    return pl.pallas_call(
        _enc_body,
        grid=(E_PAD // BE,),
        in_specs=[
            pl.BlockSpec((BE, C_), lambda i: (i, 0)),
            _full2d((C_, H_)), _full2d((1, H_)),
        ],
        out_specs=pl.BlockSpec((BE, H_), lambda i: (i, 0)),
        out_shape=jax.ShapeDtypeStruct((E_PAD, H_), F32),
    )(eap, a, c0)


def _h0_call(x, w, b):
    return pl.pallas_call(
        _h0_body,
        out_shape=jax.ShapeDtypeStruct((N_, H_), F32),
    )(x, w, b)


def _nspec2():
    return pl.BlockSpec((BN, H_), lambda i, j: (j, 0))


def _full2d_2(shape):
    return pl.BlockSpec(shape, lambda i, j: (0, 0))


_MLPBN_SCRATCH = [pltpu.VMEM((N_, H_), F32),
                  pltpu.VMEM((1, H_), F32), pltpu.VMEM((1, H_), F32),
                  pltpu.VMEM((2, H_), F32)]


def _mlpbn_call(h, p0, p1, w1, b1, w2, b2, g, be):
    nspec = _nspec2()
    return pl.pallas_call(
        _mlpbn_body,
        grid=(2, NBLK),
        in_specs=[nspec, nspec, nspec,
                  _full2d_2((H_, H_)), _full2d_2((1, H_)),
                  _full2d_2((H_, H_)), _full2d_2((1, H_)),
                  _full2d_2((1, H_)), _full2d_2((1, H_))],
        out_specs=nspec,
        out_shape=jax.ShapeDtypeStruct((N_, H_), F32),
        scratch_shapes=list(_MLPBN_SCRATCH),
    )(h, p0, p1, w1, b1, w2, b2, g, be)


def _mlpbn_cls_call(h, p0, p1, w1, b1, w2, b2, g, be, wa, wb):
    nspec = _nspec2()
    return pl.pallas_call(
        _mlpbn_cls_body,
        grid=(2, NBLK),
        in_specs=[nspec, nspec, nspec,
                  _full2d_2((H_, H_)), _full2d_2((1, H_)),
                  _full2d_2((H_, H_)), _full2d_2((1, H_)),
                  _full2d_2((1, H_)), _full2d_2((1, H_)),
                  _full2d_2((H_, H_)), _full2d_2((H_, H_))],
        out_specs=[nspec, nspec, nspec],
        out_shape=[jax.ShapeDtypeStruct((N_, H_), F32),
                   jax.ShapeDtypeStruct((N_, H_), F32),
                   jax.ShapeDtypeStruct((N_, H_), F32)],
        scratch_shapes=list(_MLPBN_SCRATCH),
    )(h, p0, p1, w1, b1, w2, b2, g, be, wa, wb)


def _head_call(s, eap, a2, c02, w, b):
    return pl.pallas_call(
        _head_body,
        grid=(E_PAD // BE,),
        in_specs=[pl.BlockSpec((BE, H_), lambda i: (i, 0)),
                  pl.BlockSpec((BE, C_), lambda i: (i, 0)),
                  _full2d((C_, H_)), _full2d((1, H_)),
                  _full2d((H_, 8)), _full2d((1, 8))],
        out_specs=pl.BlockSpec((BE, 8), lambda i: (i, 0)),
        out_shape=jax.ShapeDtypeStruct((E_PAD, 8), F32),
    )(s, eap, a2, c02, w, b)


def _sc_mesh():
    return plsc.VectorSubcoreMesh(core_axis_name="c", subcore_axis_name="s")


def _agg_partials(h, e, srcp, dstp, zrows):
    @functools.partial(
        pl.kernel,
        out_type=jax.ShapeDtypeStruct((NSC, NPAD, H_), F32),
        mesh=_sc_mesh(),
        scratch_types=[
            (pltpu.VMEM((CHA,), jnp.int32),) * 2,
            (pltpu.VMEM((CHA,), jnp.int32),) * 2,
            (pltpu.VMEM((CHA,), jnp.int32),) * 2,
            (pltpu.VMEM((CHA, H_), F32),) * 2,
            (pltpu.VMEM((CHA, H_), F32),) * 2,
            pltpu.VMEM_SHARED((NPAD, H_), F32),
            (pltpu.SemaphoreType.DMA,) * 2,
            (pltpu.SemaphoreType.DMA,) * 2,
            (pltpu.SemaphoreType.DMA,) * 2,
            (pltpu.SemaphoreType.DMA,) * 2,
        ],
    )
    def k(h_hbm, e_hbm, src_hbm, dst_hbm, z_hbm, out_hbm,
          src_v, dst_v, dst_s, rows_v, e_v, acc_sh, isem, gsem, esem, ssem):
        cid = lax.axis_index("c")
        sid = lax.axis_index("s")
        wid = cid * NTILE + sid

        def issue_idx(c, b):
            base = wid * EPW + c * CHA
            pltpu.async_copy(src_hbm.at[pl.ds(base, CHA)], src_v[b], isem[b])
            pltpu.async_copy(dst_hbm.at[pl.ds(base, CHA)], dst_v[b], isem[b])

        def drain_idx(b):
            pltpu.make_async_copy(src_hbm.at[pl.ds(0, CHA)], src_v[b], isem[b]).wait()
            pltpu.make_async_copy(dst_hbm.at[pl.ds(0, CHA)], dst_v[b], isem[b]).wait()

        def issue_main(c, b):
            base = wid * EPW + c * CHA
            pltpu.async_copy(h_hbm.at[src_v[b]], rows_v[b], gsem[b])
            pltpu.async_copy(e_hbm.at[pl.ds(base, CHA)], e_v[b], esem[b])

        def drain_main(b):
            pltpu.make_async_copy(e_hbm.at[pl.ds(0, CHA)], rows_v[b], gsem[b]).wait()
            pltpu.make_async_copy(e_hbm.at[pl.ds(0, CHA)], e_v[b], esem[b]).wait()

        def drain_scatter(b):
            pltpu.make_async_copy(e_hbm.at[pl.ds(0, CHA)], rows_v[b], ssem[b]).wait()

        pltpu.sync_copy(z_hbm.at[pl.ds(sid * NROW, NROW)],
                        acc_sh.at[pl.ds(sid * NROW, NROW)])
        plsc.subcore_barrier()

        issue_idx(0, 0)
        drain_idx(0)
        issue_main(0, 0)
        issue_idx(1, 1)

        def pair(g, carry):
            for b in range(2):
                c = 2 * g + b
                nb = 1 - b

                @pl.when(c >= 1)
                def _():
                    drain_scatter(nb)

                @pl.when(c + 1 < NCHUNKA)
                def _():
                    drain_idx(nb)
                    issue_main(c + 1, nb)

                drain_main(b)

                for j in range(CHA // 16):
                    s = pl.ds(j * 16, 16)
                    dst_s[b][s] = dst_v[b][s]

                @pl.when(c + 2 < NCHUNKA)
                def _():
                    issue_idx(c + 2, b)

                def row(r, c2):
                    for j in range(H_ // 16):
                        s = pl.ds(j * 16, 16)
                        rows_v[b][r, s] = jnp.maximum(rows_v[b][r, s] + e_v[b][r, s], 0.0)
                    return c2

                lax.fori_loop(0, CHA, row, 0, unroll=4)
                pltpu.async_copy(rows_v[b], acc_sh.at[dst_s[b]], ssem[b], add=True)
            return carry

        lax.fori_loop(0, NCHUNKA // 2, pair, 0)
        drain_scatter(1)
        plsc.subcore_barrier()
        pltpu.sync_copy(acc_sh.at[pl.ds(sid * NROW, NROW)],
                        out_hbm.at[cid, pl.ds(sid * NROW, NROW)])

    return k(h, e, srcp, dstp, zrows)


def _gather_sum(hs1, hd1, srcp, dstp):
    @functools.partial(
        pl.kernel,
        out_type=jax.ShapeDtypeStruct((E_PAD, H_), F32),
        mesh=_sc_mesh(),
        scratch_types=[
            (pltpu.VMEM((CH,), jnp.int32),) * 2,
            (pltpu.VMEM((CH,), jnp.int32),) * 2,
            (pltpu.VMEM((CH, H_), F32),) * 2,
            (pltpu.VMEM((CH, H_), F32),) * 2,
            (pltpu.SemaphoreType.DMA,) * 2,
            (pltpu.SemaphoreType.DMA,) * 2,
            (pltpu.SemaphoreType.DMA,) * 2,
            (pltpu.SemaphoreType.DMA,) * 2,
        ],
    )
    def k(hs_hbm, hd_hbm, src_hbm, dst_hbm, t_hbm,
          src_v, dst_v, a_v, b_v, isem, gasem, gbsem, wsem):
        cid = lax.axis_index("c")
        sid = lax.axis_index("s")
        wid = cid * NTILE + sid

        def issue_idx(c, b):
            base = wid * EPW + c * CH
            pltpu.async_copy(src_hbm.at[pl.ds(base, CH)], src_v[b], isem[b])
            pltpu.async_copy(dst_hbm.at[pl.ds(base, CH)], dst_v[b], isem[b])

        def drain_idx(b):
            pltpu.make_async_copy(src_hbm.at[pl.ds(0, CH)], src_v[b], isem[b]).wait()
            pltpu.make_async_copy(dst_hbm.at[pl.ds(0, CH)], dst_v[b], isem[b]).wait()

        def issue_main(c, b):
            pltpu.async_copy(hs_hbm.at[src_v[b]], a_v[b], gasem[b])
            pltpu.async_copy(hd_hbm.at[dst_v[b]], b_v[b], gbsem[b])

        def drain_main(b):
            pltpu.make_async_copy(hs_hbm.at[pl.ds(0, CH)], a_v[b], gasem[b]).wait()
            pltpu.make_async_copy(hs_hbm.at[pl.ds(0, CH)], b_v[b], gbsem[b]).wait()

        def drain_write(b):
            pltpu.make_async_copy(t_hbm.at[pl.ds(0, CH)], a_v[b], wsem[b]).wait()

        issue_idx(0, 0)
        drain_idx(0)
        issue_main(0, 0)
        issue_idx(1, 1)

        def pair(g, carry):
            for b in range(2):
                c = 2 * g + b
                nb = 1 - b

                @pl.when(c >= 1)
                def _():
                    drain_write(nb)

                @pl.when(c + 1 < NCHUNK)
                def _():
                    drain_idx(nb)
                    issue_main(c + 1, nb)

                drain_main(b)

                @pl.when(c + 2 < NCHUNK)
                def _():
                    issue_idx(c + 2, b)

                def row(r, c2):
                    for j in range(H_ // 16):
                        s = pl.ds(j * 16, 16)
                        a_v[b][r, s] = a_v[b][r, s] + b_v[b][r, s]
                    return c2

                lax.fori_loop(0, CH, row, 0, unroll=4)
                base = wid * EPW + c * CH
                pltpu.async_copy(a_v[b], t_hbm.at[pl.ds(base, CH)], wsem[b])
            return carry

        lax.fori_loop(0, NCHUNK // 2, pair, 0)
        drain_write(1)

    return k(hs1, hd1, srcp, dstp)


def kernel(x, edge_attr, W_enc, b_enc, W_node, b_node, W_edge, b_edge,
           W1, b1, W2, b2, gamma, beta, Wc1, bc1, Wc2, bc2, edge_index):
    ei = edge_index.astype(jnp.int32)
    pad = E_PAD - E_
    srcp = jnp.concatenate([ei[0], jnp.zeros((pad,), jnp.int32)])
    dstp = jnp.concatenate([ei[1], jnp.zeros((pad,), jnp.int32)])
    eap = jnp.concatenate([edge_attr.astype(F32), jnp.zeros((pad, C_), F32)], axis=0)

    w3 = W_edge.reshape(C_, H_, H_)
    a = jnp.einsum("ch,chk->ck", W_enc, w3, precision=lax.Precision.HIGHEST)
    c0 = jnp.einsum("ch,chk->k", b_enc, w3, precision=lax.Precision.HIGHEST) + b_edge
    wa = Wc1[:H_]
    wb = Wc1[H_:2 * H_]
    wcc = Wc1[2 * H_:]
    a2 = jnp.matmul(a, wcc, precision=lax.Precision.HIGHEST)
    c02 = jnp.matmul(c0, wcc, precision=lax.Precision.HIGHEST) + bc1
    w2p = jnp.zeros((H_, 8), F32).at[:, :NCLS].set(Wc2)
    b2p = jnp.zeros((1, 8), F32).at[0, :NCLS].set(bc2)
    zrows = jnp.zeros((NPAD, H_), F32)

    e = _enc_call(eap, a, c0.reshape(1, H_))
    h = _h0_call(x, W_node, b_node.reshape(1, H_))

    hs1 = hd1 = None
    for l in range(L_):
        parts = _agg_partials(h, e, srcp, dstp, zrows)[:, :N_, :]
        largs = (h, parts[0], parts[1], W1[l], b1[l].reshape(1, H_),
                 W2[l], b2[l].reshape(1, H_),
                 gamma[l].reshape(1, H_), beta[l].reshape(1, H_))
        if l < L_ - 1:
            h = _mlpbn_call(*largs)
        else:
            h, hs1, hd1 = _mlpbn_cls_call(*largs, wa, wb)

    s = _gather_sum(hs1, hd1, srcp, dstp)
    o = _head_call(s, eap, a2, c02.reshape(1, H_), w2p, b2p)
    return o[:E_, :NCLS]


# speedup vs baseline: 1.0112x; 1.0112x over previous
import functools

import jax
import jax.numpy as jnp
from jax import lax
from jax.experimental import pallas as pl
from jax.experimental.pallas import tpu as pltpu
from jax.experimental.pallas import tpu_sc as plsc

N_ = 10000
E_ = 160000
C_ = 4
H_ = 128
NCLS = 2
L_ = 2

NSC = 2
NTILE = 16
NW = NSC * NTILE
CH = 128
E_PAD = 163840
EPW = E_PAD // NW
NCHUNK = EPW // CH
CHA = 64
NCHUNKA = EPW // CHA
NPAD = 10240
NROW = NPAD // NTILE

BE = 4096
BN = 2000
F32 = jnp.float32


def _k4_affine(ea, a_ref, c0_ref):
    e = c0_ref[...]
    for c in range(C_):
        e = e + ea[:, c:c + 1] * a_ref[c:c + 1, :]
    return e


def _enc_body(ea_ref, a_ref, c0_ref, e_ref):
    i = pl.program_id(0)
    e = _k4_affine(ea_ref[...], a_ref, c0_ref)
    rows = lax.broadcasted_iota(jnp.int32, (BE, 1), 0) + i * BE
    valid = rows < E_
    e_ref[...] = jnp.where(valid, e, -1e30)


def _h0_body(x_ref, w_ref, b_ref, h_ref):
    h_ref[...] = x_ref[...] * w_ref[...] + b_ref[...]


def _mlp_body(h_ref, p0_ref, p1_ref, w1_ref, b1_ref, w2_ref, b2_ref,
              z2_ref, st_ref, s1_acc, s2_acc):
    i = pl.program_id(0)
    z = h_ref[...] + p0_ref[...] + p1_ref[...]
    y = jnp.maximum(jnp.dot(z, w1_ref[...], preferred_element_type=F32, precision=lax.Precision.HIGHEST) + b1_ref[...], 0.0)
    z2 = jnp.dot(y, w2_ref[...], preferred_element_type=F32, precision=lax.Precision.HIGHEST) + b2_ref[...]
    z2_ref[...] = z2

    @pl.when(i == 0)
    def _():
        s1_acc[...] = jnp.zeros_like(s1_acc)
        s2_acc[...] = jnp.zeros_like(s2_acc)

    s1_acc[...] = s1_acc[...] + jnp.sum(z2, axis=0, keepdims=True)
    s2_acc[...] = s2_acc[...] + jnp.sum(z2 * z2, axis=0, keepdims=True)

    @pl.when(i == (N_ // BN) - 1)
    def _():
        mu = s1_acc[...] / N_
        var = s2_acc[...] / N_ - mu * mu
        st_ref[0:1, :] = mu
        st_ref[1:2, :] = lax.rsqrt(var + 1e-5)


def _bn_body(h_ref, z2_ref, st_ref, g_ref, be_ref, ho_ref):
    mu = st_ref[0:1, :]
    rstd = st_ref[1:2, :]
    zn = (z2_ref[...] - mu) * rstd * g_ref[...] + be_ref[...]
    ho_ref[...] = h_ref[...] + jnp.maximum(zn, 0.0)


def _bn_cls_body(h_ref, z2_ref, st_ref, g_ref, be_ref, wa_ref, wb_ref,
                 ho_ref, hs_ref, hd_ref):
    mu = st_ref[0:1, :]
    rstd = st_ref[1:2, :]
    zn = (z2_ref[...] - mu) * rstd * g_ref[...] + be_ref[...]
    ho = h_ref[...] + jnp.maximum(zn, 0.0)
    ho_ref[...] = ho
    hs_ref[...] = jnp.dot(ho, wa_ref[...], preferred_element_type=F32, precision=lax.Precision.HIGHEST)
    hd_ref[...] = jnp.dot(ho, wb_ref[...], preferred_element_type=F32, precision=lax.Precision.HIGHEST)


def _head_body(s_ref, ea_ref, a2_ref, c02_ref, w_ref, b_ref, o_ref):
    e1 = _k4_affine(ea_ref[...], a2_ref, c02_ref)
    t = jnp.maximum(s_ref[...] + e1, 0.0)
    o_ref[...] = jnp.dot(t, w_ref[...],
                         preferred_element_type=F32, precision=lax.Precision.HIGHEST) + b_ref[...]


def _full2d(shape):
    return pl.BlockSpec(shape, lambda i: (0, 0))


def _enc_call(eap, a, c0):
    return pl.pallas_call(
        _enc_body,
        grid=(E_PAD // BE,),
        in_specs=[
            pl.BlockSpec((BE, C_), lambda i: (i, 0)),
            _full2d((C_, H_)), _full2d((1, H_)),
        ],
        out_specs=pl.BlockSpec((BE, H_), lambda i: (i, 0)),
        out_shape=jax.ShapeDtypeStruct((E_PAD, H_), F32),
    )(eap, a, c0)


def _h0_call(x, w, b):
    return pl.pallas_call(
        _h0_body,
        out_shape=jax.ShapeDtypeStruct((N_, H_), F32),
    )(x, w, b)


def _mlp_call(h, p0, p1, w1, b1, w2, b2):
    nspec = pl.BlockSpec((BN, H_), lambda i: (i, 0))
    return pl.pallas_call(
        _mlp_body,
        grid=(N_ // BN,),
        in_specs=[nspec, nspec, nspec,
                  _full2d((H_, H_)), _full2d((1, H_)),
                  _full2d((H_, H_)), _full2d((1, H_))],
        out_specs=[nspec, _full2d((8, H_))],
        out_shape=[jax.ShapeDtypeStruct((N_, H_), F32),
                   jax.ShapeDtypeStruct((8, H_), F32)],
        scratch_shapes=[pltpu.VMEM((1, H_), F32), pltpu.VMEM((1, H_), F32)],
    )(h, p0, p1, w1, b1, w2, b2)


def _bn_call(h, z2, st, g, be):
    nspec = pl.BlockSpec((BN, H_), lambda i: (i, 0))
    return pl.pallas_call(
        _bn_body,
        grid=(N_ // BN,),
        in_specs=[nspec, nspec, _full2d((8, H_)), _full2d((1, H_)), _full2d((1, H_))],
        out_specs=nspec,
        out_shape=jax.ShapeDtypeStruct((N_, H_), F32),
    )(h, z2, st, g, be)


def _bn_cls_call(h, z2, st, g, be, wa, wb):
    nspec = pl.BlockSpec((BN, H_), lambda i: (i, 0))
    return pl.pallas_call(
        _bn_cls_body,
        grid=(N_ // BN,),
        in_specs=[nspec, nspec, _full2d((8, H_)), _full2d((1, H_)), _full2d((1, H_)),
                  _full2d((H_, H_)), _full2d((H_, H_))],
        out_specs=[nspec, nspec, nspec],
        out_shape=[jax.ShapeDtypeStruct((N_, H_), F32),
                   jax.ShapeDtypeStruct((N_, H_), F32),
                   jax.ShapeDtypeStruct((N_, H_), F32)],
    )(h, z2, st, g, be, wa, wb)


def _head_call(s, eap, a2, c02, w, b):
    return pl.pallas_call(
        _head_body,
        grid=(E_PAD // BE,),
        in_specs=[pl.BlockSpec((BE, H_), lambda i: (i, 0)),
                  pl.BlockSpec((BE, C_), lambda i: (i, 0)),
                  _full2d((C_, H_)), _full2d((1, H_)),
                  _full2d((H_, 8)), _full2d((1, 8))],
        out_specs=pl.BlockSpec((BE, 8), lambda i: (i, 0)),
        out_shape=jax.ShapeDtypeStruct((E_PAD, 8), F32),
    )(s, eap, a2, c02, w, b)


def _sc_mesh():
    return plsc.VectorSubcoreMesh(core_axis_name="c", subcore_axis_name="s")


def _agg_partials(h, e, srcp, dstp, zrows):
    @functools.partial(
        pl.kernel,
        out_type=jax.ShapeDtypeStruct((NSC, NPAD, H_), F32),
        mesh=_sc_mesh(),
        scratch_types=[
            (pltpu.VMEM((CHA,), jnp.int32),) * 2,
            (pltpu.VMEM((CHA,), jnp.int32),) * 2,
            (pltpu.VMEM((CHA,), jnp.int32),) * 2,
            (pltpu.VMEM((CHA, H_), F32),) * 2,
            (pltpu.VMEM((CHA, H_), F32),) * 2,
            pltpu.VMEM_SHARED((NPAD, H_), F32),
            (pltpu.SemaphoreType.DMA,) * 2,
            (pltpu.SemaphoreType.DMA,) * 2,
            (pltpu.SemaphoreType.DMA,) * 2,
            (pltpu.SemaphoreType.DMA,) * 2,
        ],
    )
    def k(h_hbm, e_hbm, src_hbm, dst_hbm, z_hbm, out_hbm,
          src_v, dst_v, dst_s, rows_v, e_v, acc_sh, isem, gsem, esem, ssem):
        cid = lax.axis_index("c")
        sid = lax.axis_index("s")
        wid = cid * NTILE + sid

        def issue_idx(c, b):
            base = wid * EPW + c * CHA
            pltpu.async_copy(src_hbm.at[pl.ds(base, CHA)], src_v[b], isem[b])
            pltpu.async_copy(dst_hbm.at[pl.ds(base, CHA)], dst_v[b], isem[b])

        def drain_idx(b):
            pltpu.make_async_copy(src_hbm.at[pl.ds(0, CHA)], src_v[b], isem[b]).wait()
            pltpu.make_async_copy(dst_hbm.at[pl.ds(0, CHA)], dst_v[b], isem[b]).wait()

        def issue_main(c, b):
            base = wid * EPW + c * CHA
            pltpu.async_copy(h_hbm.at[src_v[b]], rows_v[b], gsem[b])
            pltpu.async_copy(e_hbm.at[pl.ds(base, CHA)], e_v[b], esem[b])

        def drain_main(b):
            pltpu.make_async_copy(e_hbm.at[pl.ds(0, CHA)], rows_v[b], gsem[b]).wait()
            pltpu.make_async_copy(e_hbm.at[pl.ds(0, CHA)], e_v[b], esem[b]).wait()

        def drain_scatter(b):
            pltpu.make_async_copy(e_hbm.at[pl.ds(0, CHA)], rows_v[b], ssem[b]).wait()

        pltpu.sync_copy(z_hbm.at[pl.ds(sid * NROW, NROW)],
                        acc_sh.at[pl.ds(sid * NROW, NROW)])
        plsc.subcore_barrier()

        issue_idx(0, 0)
        drain_idx(0)
        issue_main(0, 0)
        issue_idx(1, 1)

        def pair(g, carry):
            for b in range(2):
                c = 2 * g + b
                nb = 1 - b

                @pl.when(c >= 1)
                def _():
                    drain_scatter(nb)

                @pl.when(c + 1 < NCHUNKA)
                def _():
                    drain_idx(nb)
                    issue_main(c + 1, nb)

                drain_main(b)

                for j in range(CHA // 16):
                    s = pl.ds(j * 16, 16)
                    dst_s[b][s] = dst_v[b][s]

                @pl.when(c + 2 < NCHUNKA)
                def _():
                    issue_idx(c + 2, b)

                def row(r, c2):
                    for j in range(H_ // 16):
                        s = pl.ds(j * 16, 16)
                        rows_v[b][r, s] = jnp.maximum(rows_v[b][r, s] + e_v[b][r, s], 0.0)
                    return c2

                lax.fori_loop(0, CHA, row, 0, unroll=4)
                pltpu.async_copy(rows_v[b], acc_sh.at[dst_s[b]], ssem[b], add=True)
            return carry

        lax.fori_loop(0, NCHUNKA // 2, pair, 0)
        drain_scatter(1)
        plsc.subcore_barrier()
        pltpu.sync_copy(acc_sh.at[pl.ds(sid * NROW, NROW)],
                        out_hbm.at[cid, pl.ds(sid * NROW, NROW)])

    return k(h, e, srcp, dstp, zrows)


def _gather_sum(hs1, hd1, srcp, dstp):
    @functools.partial(
        pl.kernel,
        out_type=jax.ShapeDtypeStruct((E_PAD, H_), F32),
        mesh=_sc_mesh(),
        scratch_types=[
            (pltpu.VMEM((CH,), jnp.int32),) * 2,
            (pltpu.VMEM((CH,), jnp.int32),) * 2,
            (pltpu.VMEM((CH, H_), F32),) * 2,
            (pltpu.VMEM((CH, H_), F32),) * 2,
            (pltpu.SemaphoreType.DMA,) * 2,
            (pltpu.SemaphoreType.DMA,) * 2,
            (pltpu.SemaphoreType.DMA,) * 2,
            (pltpu.SemaphoreType.DMA,) * 2,
        ],
    )
    def k(hs_hbm, hd_hbm, src_hbm, dst_hbm, t_hbm,
          src_v, dst_v, a_v, b_v, isem, gasem, gbsem, wsem):
        cid = lax.axis_index("c")
        sid = lax.axis_index("s")
        wid = cid * NTILE + sid

        def issue_idx(c, b):
            base = wid * EPW + c * CH
            pltpu.async_copy(src_hbm.at[pl.ds(base, CH)], src_v[b], isem[b])
            pltpu.async_copy(dst_hbm.at[pl.ds(base, CH)], dst_v[b], isem[b])

        def drain_idx(b):
            pltpu.make_async_copy(src_hbm.at[pl.ds(0, CH)], src_v[b], isem[b]).wait()
            pltpu.make_async_copy(dst_hbm.at[pl.ds(0, CH)], dst_v[b], isem[b]).wait()

        def issue_main(c, b):
            pltpu.async_copy(hs_hbm.at[src_v[b]], a_v[b], gasem[b])
            pltpu.async_copy(hd_hbm.at[dst_v[b]], b_v[b], gbsem[b])

        def drain_main(b):
            pltpu.make_async_copy(hs_hbm.at[pl.ds(0, CH)], a_v[b], gasem[b]).wait()
            pltpu.make_async_copy(hs_hbm.at[pl.ds(0, CH)], b_v[b], gbsem[b]).wait()

        def drain_write(b):
            pltpu.make_async_copy(t_hbm.at[pl.ds(0, CH)], a_v[b], wsem[b]).wait()

        issue_idx(0, 0)
        drain_idx(0)
        issue_main(0, 0)
        issue_idx(1, 1)

        def pair(g, carry):
            for b in range(2):
                c = 2 * g + b
                nb = 1 - b

                @pl.when(c >= 1)
                def _():
                    drain_write(nb)

                @pl.when(c + 1 < NCHUNK)
                def _():
                    drain_idx(nb)
                    issue_main(c + 1, nb)

                drain_main(b)

                @pl.when(c + 2 < NCHUNK)
                def _():
                    issue_idx(c + 2, b)

                def row(r, c2):
                    for j in range(H_ // 16):
                        s = pl.ds(j * 16, 16)
                        a_v[b][r, s] = a_v[b][r, s] + b_v[b][r, s]
                    return c2

                lax.fori_loop(0, CH, row, 0, unroll=4)
                base = wid * EPW + c * CH
                pltpu.async_copy(a_v[b], t_hbm.at[pl.ds(base, CH)], wsem[b])
            return carry

        lax.fori_loop(0, NCHUNK // 2, pair, 0)
        drain_write(1)

    return k(hs1, hd1, srcp, dstp)


def kernel(x, edge_attr, W_enc, b_enc, W_node, b_node, W_edge, b_edge,
           W1, b1, W2, b2, gamma, beta, Wc1, bc1, Wc2, bc2, edge_index):
    ei = edge_index.astype(jnp.int32)
    pad = E_PAD - E_
    srcp = jnp.concatenate([ei[0], jnp.zeros((pad,), jnp.int32)])
    dstp = jnp.concatenate([ei[1], jnp.zeros((pad,), jnp.int32)])
    eap = jnp.concatenate([edge_attr.astype(F32), jnp.zeros((pad, C_), F32)], axis=0)

    w3 = W_edge.reshape(C_, H_, H_)
    a = jnp.einsum("ch,chk->ck", W_enc, w3, precision=lax.Precision.HIGHEST)
    c0 = jnp.einsum("ch,chk->k", b_enc, w3, precision=lax.Precision.HIGHEST) + b_edge
    wa = Wc1[:H_]
    wb = Wc1[H_:2 * H_]
    wcc = Wc1[2 * H_:]
    a2 = jnp.matmul(a, wcc, precision=lax.Precision.HIGHEST)
    c02 = jnp.matmul(c0, wcc, precision=lax.Precision.HIGHEST) + bc1
    w2p = jnp.zeros((H_, 8), F32).at[:, :NCLS].set(Wc2)
    b2p = jnp.zeros((1, 8), F32).at[0, :NCLS].set(bc2)
    zrows = jnp.zeros((NPAD, H_), F32)

    e = _enc_call(eap, a, c0.reshape(1, H_))
    h = _h0_call(x, W_node, b_node.reshape(1, H_))

    hs1 = hd1 = None
    for l in range(L_):
        parts = _agg_partials(h, e, srcp, dstp, zrows)[:, :N_, :]
        z2, st = _mlp_call(h, parts[0], parts[1], W1[l], b1[l].reshape(1, H_),
                           W2[l], b2[l].reshape(1, H_))
        if l < L_ - 1:
            h = _bn_call(h, z2, st, gamma[l].reshape(1, H_), beta[l].reshape(1, H_))
        else:
            h, hs1, hd1 = _bn_cls_call(h, z2, st, gamma[l].reshape(1, H_),
                                       beta[l].reshape(1, H_), wa, wb)

    s = _gather_sum(hs1, hd1, srcp, dstp)
    o = _head_call(s, eap, a2, c02.reshape(1, H_), w2p, b2p)
    return o[:E_, :NCLS]


# speedup vs baseline: 1.1421x; 1.1295x over previous
import functools

import jax
import jax.numpy as jnp
from jax import lax
from jax.experimental import pallas as pl
from jax.experimental.pallas import tpu as pltpu
from jax.experimental.pallas import tpu_sc as plsc

N_ = 10000
E_ = 160000
C_ = 4
H_ = 128
NCLS = 2
L_ = 2

NSC = 2
NTILE = 16
NW = NSC * NTILE
CH = 128
E_PAD = 163840
EPW = E_PAD // NW
NCHUNK = EPW // CH
CHA = 64
NCHUNKA = EPW // CHA
NPAD = 10240
NROW = NPAD // NTILE

BE = 4096
BN = 2000
F32 = jnp.float32


def _k4_affine(ea, a_ref, c0_ref):
    e = c0_ref[...]
    for c in range(C_):
        e = e + ea[:, c:c + 1] * a_ref[c:c + 1, :]
    return e


def _enc_body(ea_ref, a_ref, c0_ref, e_ref):
    i = pl.program_id(0)
    e = _k4_affine(ea_ref[...], a_ref, c0_ref)
    rows = lax.broadcasted_iota(jnp.int32, (BE, 1), 0) + i * BE
    valid = rows < E_
    e_ref[...] = jnp.where(valid, e, -1e30)


def _h0_body(x_ref, w_ref, b_ref, h_ref):
    h_ref[...] = x_ref[...] * w_ref[...] + b_ref[...]


def _mlp_body(h_ref, p0_ref, p1_ref, w1_ref, b1_ref, w2_ref, b2_ref,
              z2_ref, st_ref, s1_acc, s2_acc):
    i = pl.program_id(0)
    z = h_ref[...] + p0_ref[...] + p1_ref[...]
    y = jnp.maximum(jnp.dot(z, w1_ref[...], preferred_element_type=F32, precision=lax.Precision.HIGHEST) + b1_ref[...], 0.0)
    z2 = jnp.dot(y, w2_ref[...], preferred_element_type=F32, precision=lax.Precision.HIGHEST) + b2_ref[...]
    z2_ref[...] = z2

    @pl.when(i == 0)
    def _():
        s1_acc[...] = jnp.zeros_like(s1_acc)
        s2_acc[...] = jnp.zeros_like(s2_acc)

    s1_acc[...] = s1_acc[...] + jnp.sum(z2, axis=0, keepdims=True)
    s2_acc[...] = s2_acc[...] + jnp.sum(z2 * z2, axis=0, keepdims=True)

    @pl.when(i == (N_ // BN) - 1)
    def _():
        mu = s1_acc[...] / N_
        var = s2_acc[...] / N_ - mu * mu
        st_ref[0:1, :] = mu
        st_ref[1:2, :] = lax.rsqrt(var + 1e-5)


def _bn_body(h_ref, z2_ref, st_ref, g_ref, be_ref, ho_ref):
    mu = st_ref[0:1, :]
    rstd = st_ref[1:2, :]
    zn = (z2_ref[...] - mu) * rstd * g_ref[...] + be_ref[...]
    ho_ref[...] = h_ref[...] + jnp.maximum(zn, 0.0)


def _bn_cls_body(h_ref, z2_ref, st_ref, g_ref, be_ref, wa_ref, wb_ref,
                 ho_ref, hs_ref, hd_ref):
    mu = st_ref[0:1, :]
    rstd = st_ref[1:2, :]
    zn = (z2_ref[...] - mu) * rstd * g_ref[...] + be_ref[...]
    ho = h_ref[...] + jnp.maximum(zn, 0.0)
    ho_ref[...] = ho
    hs_ref[...] = jnp.dot(ho, wa_ref[...], preferred_element_type=F32, precision=lax.Precision.HIGHEST)
    hd_ref[...] = jnp.dot(ho, wb_ref[...], preferred_element_type=F32, precision=lax.Precision.HIGHEST)


def _head_body(s_ref, ea_ref, a2_ref, c02_ref, w_ref, b_ref, o_ref):
    e1 = _k4_affine(ea_ref[...], a2_ref, c02_ref)
    t = jnp.maximum(s_ref[...] + e1, 0.0)
    o = jnp.dot(t, w_ref[...],
                preferred_element_type=F32, precision=lax.Precision.HIGHEST) + b_ref[...]
    o_ref[...] = o[:, :NCLS]


def _full2d(shape):
    return pl.BlockSpec(shape, lambda i: (0, 0))


def _enc_call(eap, a, c0):
    return pl.pallas_call(
        _enc_body,
        grid=(E_PAD // BE,),
        in_specs=[
            pl.BlockSpec((BE, C_), lambda i: (i, 0)),
            _full2d((C_, H_)), _full2d((1, H_)),
        ],
        out_specs=pl.BlockSpec((BE, H_), lambda i: (i, 0)),
        out_shape=jax.ShapeDtypeStruct((E_PAD, H_), F32),
    )(eap, a, c0)


def _h0_call(x, w, b):
    return pl.pallas_call(
        _h0_body,
        out_shape=jax.ShapeDtypeStruct((N_, H_), F32),
    )(x, w, b)


def _mlp_call(h, p0, p1, w1, b1, w2, b2):
    nspec = pl.BlockSpec((BN, H_), lambda i: (i, 0))
    return pl.pallas_call(
        _mlp_body,
        grid=(N_ // BN,),
        in_specs=[nspec, nspec, nspec,
                  _full2d((H_, H_)), _full2d((1, H_)),
                  _full2d((H_, H_)), _full2d((1, H_))],
        out_specs=[nspec, _full2d((8, H_))],
        out_shape=[jax.ShapeDtypeStruct((N_, H_), F32),
                   jax.ShapeDtypeStruct((8, H_), F32)],
        scratch_shapes=[pltpu.VMEM((1, H_), F32), pltpu.VMEM((1, H_), F32)],
    )(h, p0, p1, w1, b1, w2, b2)


def _bn_call(h, z2, st, g, be):
    nspec = pl.BlockSpec((BN, H_), lambda i: (i, 0))
    return pl.pallas_call(
        _bn_body,
        grid=(N_ // BN,),
        in_specs=[nspec, nspec, _full2d((8, H_)), _full2d((1, H_)), _full2d((1, H_))],
        out_specs=nspec,
        out_shape=jax.ShapeDtypeStruct((N_, H_), F32),
    )(h, z2, st, g, be)


def _bn_cls_call(h, z2, st, g, be, wa, wb):
    nspec = pl.BlockSpec((BN, H_), lambda i: (i, 0))
    return pl.pallas_call(
        _bn_cls_body,
        grid=(N_ // BN,),
        in_specs=[nspec, nspec, _full2d((8, H_)), _full2d((1, H_)), _full2d((1, H_)),
                  _full2d((H_, H_)), _full2d((H_, H_))],
        out_specs=[nspec, nspec, nspec],
        out_shape=[jax.ShapeDtypeStruct((N_, H_), F32),
                   jax.ShapeDtypeStruct((N_, H_), F32),
                   jax.ShapeDtypeStruct((N_, H_), F32)],
    )(h, z2, st, g, be, wa, wb)


def _head_call(s, eaf, a2, c02, w, b):
    return pl.pallas_call(
        _head_body,
        grid=(E_PAD // BE,),
        in_specs=[pl.BlockSpec((BE, H_), lambda i: (i, 0)),
                  pl.BlockSpec((BE, C_), lambda i: (i, 0)),
                  _full2d((C_, H_)), _full2d((1, H_)),
                  _full2d((H_, 8)), _full2d((1, 8))],
        out_specs=pl.BlockSpec((BE, NCLS), lambda i: (i, 0)),
        out_shape=jax.ShapeDtypeStruct((E_, NCLS), F32),
    )(s, eaf, a2, c02, w, b)


def _sc_mesh():
    return plsc.VectorSubcoreMesh(core_axis_name="c", subcore_axis_name="s")


def _agg_partials(h, e, srcp, dstp, zrows):
    @functools.partial(
        pl.kernel,
        out_type=jax.ShapeDtypeStruct((NSC, NPAD, H_), F32),
        mesh=_sc_mesh(),
        scratch_types=[
            (pltpu.VMEM((CHA,), jnp.int32),) * 2,
            (pltpu.VMEM((CHA,), jnp.int32),) * 2,
            (pltpu.VMEM((CHA,), jnp.int32),) * 2,
            (pltpu.VMEM((CHA, H_), F32),) * 2,
            (pltpu.VMEM((CHA, H_), F32),) * 2,
            pltpu.VMEM_SHARED((NPAD, H_), F32),
            (pltpu.SemaphoreType.DMA,) * 2,
            (pltpu.SemaphoreType.DMA,) * 2,
            (pltpu.SemaphoreType.DMA,) * 2,
            (pltpu.SemaphoreType.DMA,) * 2,
        ],
    )
    def k(h_hbm, e_hbm, src_hbm, dst_hbm, z_hbm, out_hbm,
          src_v, dst_v, dst_s, rows_v, e_v, acc_sh, isem, gsem, esem, ssem):
        cid = lax.axis_index("c")
        sid = lax.axis_index("s")
        wid = cid * NTILE + sid

        def issue_idx(c, b):
            base = wid * EPW + c * CHA
            pltpu.async_copy(src_hbm.at[pl.ds(base, CHA)], src_v[b], isem[b])
            pltpu.async_copy(dst_hbm.at[pl.ds(base, CHA)], dst_v[b], isem[b])

        def drain_idx(b):
            pltpu.make_async_copy(src_hbm.at[pl.ds(0, CHA)], src_v[b], isem[b]).wait()
            pltpu.make_async_copy(dst_hbm.at[pl.ds(0, CHA)], dst_v[b], isem[b]).wait()

        def issue_main(c, b):
            base = wid * EPW + c * CHA
            pltpu.async_copy(h_hbm.at[src_v[b]], rows_v[b], gsem[b])
            pltpu.async_copy(e_hbm.at[pl.ds(base, CHA)], e_v[b], esem[b])

        def drain_main(b):
            pltpu.make_async_copy(e_hbm.at[pl.ds(0, CHA)], rows_v[b], gsem[b]).wait()
            pltpu.make_async_copy(e_hbm.at[pl.ds(0, CHA)], e_v[b], esem[b]).wait()

        def drain_scatter(b):
            pltpu.make_async_copy(e_hbm.at[pl.ds(0, CHA)], rows_v[b], ssem[b]).wait()

        pltpu.sync_copy(z_hbm.at[pl.ds(sid * NROW, NROW)],
                        acc_sh.at[pl.ds(sid * NROW, NROW)])
        plsc.subcore_barrier()

        issue_idx(0, 0)
        drain_idx(0)
        issue_main(0, 0)
        issue_idx(1, 1)

        def pair(g, carry):
            for b in range(2):
                c = 2 * g + b
                nb = 1 - b

                @pl.when(c >= 1)
                def _():
                    drain_scatter(nb)

                @pl.when(c + 1 < NCHUNKA)
                def _():
                    drain_idx(nb)
                    issue_main(c + 1, nb)

                drain_main(b)

                for j in range(CHA // 16):
                    s = pl.ds(j * 16, 16)
                    dst_s[b][s] = dst_v[b][s]

                @pl.when(c + 2 < NCHUNKA)
                def _():
                    issue_idx(c + 2, b)

                def row(r, c2):
                    for j in range(H_ // 16):
                        s = pl.ds(j * 16, 16)
                        rows_v[b][r, s] = jnp.maximum(rows_v[b][r, s] + e_v[b][r, s], 0.0)
                    return c2

                lax.fori_loop(0, CHA, row, 0, unroll=4)
                pltpu.async_copy(rows_v[b], acc_sh.at[dst_s[b]], ssem[b], add=True)
            return carry

        lax.fori_loop(0, NCHUNKA // 2, pair, 0)
        drain_scatter(1)
        plsc.subcore_barrier()
        pltpu.sync_copy(acc_sh.at[pl.ds(sid * NROW, NROW)],
                        out_hbm.at[cid, pl.ds(sid * NROW, NROW)])

    return k(h, e, srcp, dstp, zrows)


def _gather_sum(hs1, hd1, srcp, dstp):
    @functools.partial(
        pl.kernel,
        out_type=jax.ShapeDtypeStruct((E_PAD, H_), F32),
        mesh=_sc_mesh(),
        scratch_types=[
            (pltpu.VMEM((CH,), jnp.int32),) * 2,
            (pltpu.VMEM((CH,), jnp.int32),) * 2,
            (pltpu.VMEM((CH, H_), F32),) * 2,
            (pltpu.VMEM((CH, H_), F32),) * 2,
            (pltpu.SemaphoreType.DMA,) * 2,
            (pltpu.SemaphoreType.DMA,) * 2,
            (pltpu.SemaphoreType.DMA,) * 2,
            (pltpu.SemaphoreType.DMA,) * 2,
        ],
    )
    def k(hs_hbm, hd_hbm, src_hbm, dst_hbm, t_hbm,
          src_v, dst_v, a_v, b_v, isem, gasem, gbsem, wsem):
        cid = lax.axis_index("c")
        sid = lax.axis_index("s")
        wid = cid * NTILE + sid

        def issue_idx(c, b):
            base = wid * EPW + c * CH
            pltpu.async_copy(src_hbm.at[pl.ds(base, CH)], src_v[b], isem[b])
            pltpu.async_copy(dst_hbm.at[pl.ds(base, CH)], dst_v[b], isem[b])

        def drain_idx(b):
            pltpu.make_async_copy(src_hbm.at[pl.ds(0, CH)], src_v[b], isem[b]).wait()
            pltpu.make_async_copy(dst_hbm.at[pl.ds(0, CH)], dst_v[b], isem[b]).wait()

        def issue_main(c, b):
            pltpu.async_copy(hs_hbm.at[src_v[b]], a_v[b], gasem[b])
            pltpu.async_copy(hd_hbm.at[dst_v[b]], b_v[b], gbsem[b])

        def drain_main(b):
            pltpu.make_async_copy(hs_hbm.at[pl.ds(0, CH)], a_v[b], gasem[b]).wait()
            pltpu.make_async_copy(hs_hbm.at[pl.ds(0, CH)], b_v[b], gbsem[b]).wait()

        def drain_write(b):
            pltpu.make_async_copy(t_hbm.at[pl.ds(0, CH)], a_v[b], wsem[b]).wait()

        issue_idx(0, 0)
        drain_idx(0)
        issue_main(0, 0)
        issue_idx(1, 1)

        def pair(g, carry):
            for b in range(2):
                c = 2 * g + b
                nb = 1 - b

                @pl.when(c >= 1)
                def _():
                    drain_write(nb)

                @pl.when(c + 1 < NCHUNK)
                def _():
                    drain_idx(nb)
                    issue_main(c + 1, nb)

                drain_main(b)

                @pl.when(c + 2 < NCHUNK)
                def _():
                    issue_idx(c + 2, b)

                def row(r, c2):
                    for j in range(H_ // 16):
                        s = pl.ds(j * 16, 16)
                        a_v[b][r, s] = a_v[b][r, s] + b_v[b][r, s]
                    return c2

                lax.fori_loop(0, CH, row, 0, unroll=4)
                base = wid * EPW + c * CH
                pltpu.async_copy(a_v[b], t_hbm.at[pl.ds(base, CH)], wsem[b])
            return carry

        lax.fori_loop(0, NCHUNK // 2, pair, 0)
        drain_write(1)

    return k(hs1, hd1, srcp, dstp)


def kernel(x, edge_attr, W_enc, b_enc, W_node, b_node, W_edge, b_edge,
           W1, b1, W2, b2, gamma, beta, Wc1, bc1, Wc2, bc2, edge_index):
    ei = edge_index.astype(jnp.int32)
    pad = E_PAD - E_
    srcp = jnp.concatenate([ei[0], jnp.zeros((pad,), jnp.int32)])
    dstp = jnp.concatenate([ei[1], jnp.zeros((pad,), jnp.int32)])
    eaf = edge_attr.astype(F32)

    w3 = W_edge.reshape(C_, H_, H_)
    a = jnp.einsum("ch,chk->ck", W_enc, w3, precision=lax.Precision.HIGHEST)
    c0 = jnp.einsum("ch,chk->k", b_enc, w3, precision=lax.Precision.HIGHEST) + b_edge
    wa = Wc1[:H_]
    wb = Wc1[H_:2 * H_]
    wcc = Wc1[2 * H_:]
    a2 = jnp.matmul(a, wcc, precision=lax.Precision.HIGHEST)
    c02 = jnp.matmul(c0, wcc, precision=lax.Precision.HIGHEST) + bc1
    w2p = jnp.zeros((H_, 8), F32).at[:, :NCLS].set(Wc2)
    b2p = jnp.zeros((1, 8), F32).at[0, :NCLS].set(bc2)
    zrows = jnp.zeros((NPAD, H_), F32)

    e = _enc_call(eaf, a, c0.reshape(1, H_))
    h = _h0_call(x, W_node, b_node.reshape(1, H_))

    hs1 = hd1 = None
    for l in range(L_):
        parts = _agg_partials(h, e, srcp, dstp, zrows)[:, :N_, :]
        z2, st = _mlp_call(h, parts[0], parts[1], W1[l], b1[l].reshape(1, H_),
                           W2[l], b2[l].reshape(1, H_))
        if l < L_ - 1:
            h = _bn_call(h, z2, st, gamma[l].reshape(1, H_), beta[l].reshape(1, H_))
        else:
            h, hs1, hd1 = _bn_cls_call(h, z2, st, gamma[l].reshape(1, H_),
                                       beta[l].reshape(1, H_), wa, wb)

    s = _gather_sum(hs1, hd1, srcp, dstp)
    return _head_call(s, eaf, a2, c02.reshape(1, H_), w2p, b2p)
